# baseline (speedup 1.0000x reference)
import jax
import jax.numpy as jnp
from jax.experimental import pallas as pl
from jax.experimental.pallas import tpu as pltpu

_G = 16
_S = 4


def _part_scan(ref, base_slab, ng, d):
    mv = jnp.full((_G, d), jnp.inf, jnp.float32)
    mi = jnp.zeros((_G, d), jnp.int32)
    for g in range(ng):
        v = ref[0, pl.ds(g * _G, _G), :]
        mask = v < mv
        mv = jnp.where(mask, v, mv)
        mi = jnp.where(mask, jnp.int32(base_slab + g), mi)
    return mv, mi


def _argmin_body(*refs):
    x_refs, o_ref = refs[:-1], refs[-1]
    nh, d = x_refs[0].shape[1], x_refs[0].shape[2]
    ng = nh // _G
    mv, mi = _part_scan(x_refs[0], 0, ng, d)
    for s in range(1, _S):
        mvs, mis = _part_scan(x_refs[s], s * ng, ng, d)
        take = mvs < mv
        mv = jnp.where(take, mvs, mv)
        mi = jnp.where(take, mis, mi)
    m = jnp.min(mv, axis=0)
    track = jax.lax.broadcasted_iota(jnp.int32, (_G, d), 0)
    full = mi * _G + track
    big = jnp.int32(2**30)
    cand = jnp.where(mv == m[None], full, big)
    o_ref[0, 0, :] = jnp.min(cand, axis=0)


def kernel(x):
    B, N, D = x.shape
    Nh = N // _S
    out = pl.pallas_call(
        _argmin_body,
        grid=(B,),
        in_specs=[
            pl.BlockSpec((1, Nh, D), lambda b, s=s: (b, s, 0))
            for s in range(_S)
        ],
        out_specs=pl.BlockSpec((1, 1, D), lambda b: (b, 0, 0)),
        out_shape=jax.ShapeDtypeStruct((B, 1, D), jnp.int32),
        compiler_params=pltpu.CompilerParams(
            dimension_semantics=("arbitrary",),
        ),
    )(*([x] * _S))
    return out.reshape(B, D).astype(jnp.int64)
